# baseline (speedup 1.0000x reference)
import functools

import jax
import jax.numpy as jnp
from jax import lax
from jax.experimental import pallas as pl
from jax.experimental.pallas import tpu as pltpu
from jax.experimental.pallas import tpu_sc as plsc

_C = 128
_NBUF = 5


@functools.lru_cache(maxsize=None)
def _build(bt, h, d):
    info = plsc.get_sparse_core_info()
    nc, ns = info.num_cores, info.num_subcores
    nw = nc * ns
    assert bt % (nw * _C) == 0
    n_groups = h // _NBUF
    assert h == n_groups * _NBUF
    mesh = plsc.VectorSubcoreMesh(core_axis_name="c", subcore_axis_name="s")

    def body(xt_hbm, tab_hbm, out_hbm, idx_v, rows_v, *sems):
        gsems, ssems = sems[:_NBUF], sems[_NBUF:]
        wid = lax.axis_index("s") * nc + lax.axis_index("c")
        base = wid * _C
        pltpu.sync_copy(xt_hbm.at[:, pl.ds(base, _C)], idx_v)

        def start_gather(j, b):
            pltpu.make_async_copy(
                tab_hbm.at[idx_v.at[j]], rows_v.at[b], gsems[b]
            ).start()

        def wait_gather(b):
            pltpu.make_async_copy(
                tab_hbm.at[idx_v.at[0]], rows_v.at[b], gsems[b]
            ).wait()

        def start_store(j, b):
            pltpu.make_async_copy(
                rows_v.at[b], out_hbm.at[j, pl.ds(base, _C)], ssems[b]
            ).start()

        def wait_store(b):
            pltpu.make_async_copy(
                rows_v.at[b], out_hbm.at[0, pl.ds(base, _C)], ssems[b]
            ).wait()

        for b in range(_NBUF):
            start_gather(b, b)

        def group(g, carry):
            for b in range(_NBUF):
                j = g * _NBUF + b
                wait_gather(b)
                start_store(j, b)

                @pl.when(g < n_groups - 1)
                def _():
                    wait_store(b)
                    start_gather(j + _NBUF, b)

            return carry

        lax.fori_loop(0, n_groups, group, 0)
        for b in range(_NBUF):
            wait_store(b)

    return pl.kernel(
        body,
        mesh=mesh,
        out_type=jax.ShapeDtypeStruct((h, bt, d), jnp.float32),
        scratch_types=[
            pltpu.VMEM((h, _C), jnp.int32),
            pltpu.VMEM((_NBUF, _C, d), jnp.float32),
        ]
        + [pltpu.SemaphoreType.DMA] * (2 * _NBUF),
        compiler_params=pltpu.CompilerParams(use_tc_tiling_on_sc=True),
    )


def kernel(x, table):
    bt, h = x.shape
    _, d = table.shape
    xt = jnp.transpose(x.astype(jnp.int32))
    out = _build(bt, h, d)(xt, table)
    return jnp.transpose(out, (1, 0, 2))
